# baseline (speedup 1.0000x reference)
import functools

import jax
import jax.numpy as jnp
from jax import lax
from jax.experimental import pallas as pl
from jax.experimental.pallas import tpu as pltpu
from jax.experimental.pallas import tpu_sc as plsc

N_WORDS = 1000000
D = 64
B = 16384
L = 16
NC = 2
NS = 16
NW = NC * NS
BPW = B // NW
CH = 128
NCH = BPW // CH
NV = D // L


def _body(wids_hbm, cids_hbm, table_hbm, params_hbm, out_hbm,
          idx_w, idx_c, wrows, crows, pvec, outv, tr, sem):
  wid = lax.axis_index("s") * NC + lax.axis_index("c")
  base = wid * BPW

  pltpu.sync_copy(wids_hbm.at[wid], idx_w)
  pltpu.sync_copy(cids_hbm.at[wid], idx_c)
  pltpu.sync_copy(params_hbm, pvec)

  copies = []
  for j in range(NCH):
    copies.append(pltpu.async_copy(
        table_hbm.at[idx_w.at[j]], wrows.at[pl.ds(j * CH, CH)], sem))
    copies.append(pltpu.async_copy(
        table_hbm.at[idx_c.at[j]], crows.at[pl.ds(j * CH, CH)], sem))
  for c in copies:
    c.wait()

  wbv = pvec[0, :]
  cbv = pvec[1, :]
  fw = [pvec[4 + i, :] for i in range(NV)]

  fcbv = pvec[2, :]
  lane = lax.broadcasted_iota(jnp.int32, (L,), 0)

  def group(g, _):
    for k in range(L):
      e = g * L + k
      acc = None
      for i in range(NV):
        w = wrows[e, pl.ds(i * L, L)]
        c = crows[e, pl.ds(i * L, L)]
        t = ((w + wbv) * (c + cbv)) * fw[i]
        acc = t if acc is None else acc + t
      plsc.store_scatter(tr, [lane, jnp.full((L,), k, jnp.int32)], acc)
    zv = None
    for j in range(L):
      r = tr[j, pl.ds(0, L)]
      zv = r if zv is None else zv + r
    x = zv + fcbv
    outv[pl.ds(g * L, L)] = 1.0 / (1.0 + jnp.exp(-x))
    return _

  lax.fori_loop(0, BPW // L, group, None)

  pltpu.sync_copy(outv, out_hbm.at[pl.ds(base, BPW)])


@jax.jit
def _run(wids3, cids3, table, params):
  mesh = plsc.VectorSubcoreMesh(
      core_axis_name="c", subcore_axis_name="s",
      num_cores=NC, num_subcores=NS)
  return pl.kernel(
      _body,
      out_type=jax.ShapeDtypeStruct((B,), jnp.float32),
      mesh=mesh,
      compiler_params=pltpu.CompilerParams(
          needs_layout_passes=False, use_tc_tiling_on_sc=False),
      scratch_types=[
          pltpu.VMEM((NCH, CH), jnp.int32),
          pltpu.VMEM((NCH, CH), jnp.int32),
          pltpu.VMEM((BPW, D), jnp.float32),
          pltpu.VMEM((BPW, D), jnp.float32),
          pltpu.VMEM((8, L), jnp.float32),
          pltpu.VMEM((BPW,), jnp.float32),
          pltpu.VMEM((L, L + 1), jnp.float32),
          pltpu.SemaphoreType.DMA,
      ],
  )(wids3, cids3, table, params)


def kernel(word_ids, context_ids, table, fc_w, fc_b, word_bias, con_bias):
  wids3 = word_ids.astype(jnp.int32).reshape(NW, NCH, CH)
  cids3 = context_ids.astype(jnp.int32).reshape(NW, NCH, CH)
  params = jnp.concatenate([
      jnp.broadcast_to(word_bias.astype(jnp.float32), (L,)),
      jnp.broadcast_to(con_bias.astype(jnp.float32), (L,)),
      jnp.broadcast_to(fc_b.astype(jnp.float32), (L,)),
      jnp.zeros((L,), jnp.float32),
      fc_w.astype(jnp.float32).reshape(D),
  ]).reshape(8, L)
  out = _run(wids3, cids3, table, params)
  return out.reshape(B, 1)
